# baseline (speedup 1.0000x reference)
import functools

import jax
import jax.numpy as jnp
from jax.experimental import pallas as pl

K_NN = 20


def _safe_norm(a, axis):
    return jnp.sqrt(jnp.sum(a * a, axis=axis) + 1e-12)


def _cos(a, b):
    num = jnp.sum(a * b, axis=-1)
    den = jnp.maximum(_safe_norm(a, -1) * _safe_norm(b, -1), 1e-8)
    return num / den


def _pdr(x, new_x):
    B, S, K, _ = x.shape
    nx = x - new_x[:, :, None, :]
    mean_x = jnp.mean(nx, axis=2, keepdims=True)
    pdist1 = _safe_norm(nx, 3)[:, None]
    pdist2 = _safe_norm(nx - mean_x, 3)[:, None]
    cos1 = _cos(nx, mean_x)[:, None]
    mean_x2 = jnp.mean(x, axis=2, keepdims=True)
    norms = x - mean_x2
    nnorms = new_x[:, :, None, :] - mean_x2
    pdist3 = _safe_norm(nx + norms - nnorms, 3)[:, None]
    cos2 = _cos(nx, norms)[:, None]
    cos3 = _cos(norms, nnorms.reshape(B, S, 1, 3))[:, None]
    out = jnp.concatenate([pdist1, pdist2, pdist3, cos1, cos2, cos3], axis=1)
    return jnp.concatenate([out, out - jnp.mean(out, axis=-1, keepdims=True)], axis=1)


def _knn(x, k):
    inner = -2.0 * jnp.einsum('bcn,bcm->bnm', x, x)
    xx = jnp.sum(x ** 2, axis=1, keepdims=True)
    pairwise = -xx - inner - jnp.transpose(xx, (0, 2, 1))
    _, idx = jax.lax.top_k(pairwise, k)
    return idx


def _bn(h, g, b):
    mean = jnp.mean(h, axis=(0, 2, 3), keepdims=True)
    var = jnp.var(h, axis=(0, 2, 3), keepdims=True)
    return g[None, :, None, None] * (h - mean) / jnp.sqrt(var + 1e-5) + b[None, :, None, None]


def _copy_kernel(x_ref, o_ref):
    o_ref[...] = x_ref[...]


def kernel(x, xyz, W1, g1, b1, W2, bias2, g2, b2):
    B = xyz.shape[0]
    N = xyz.shape[2]
    idx = _knn(xyz, K_NN)
    xyz_t = jnp.transpose(xyz, (0, 2, 1))
    batch_idx = jnp.arange(B)[:, None, None]
    x_knn = xyz_t[batch_idx, idx]
    pdx = _pdr(x_knn, xyz_t)
    h = jnp.einsum('oc,bcnk->bonk', W1, pdx)
    h = jax.nn.gelu(_bn(h, g1, b1), approximate=False)
    h2 = jnp.einsum('oc,bcnk->bonk', W2, h) + bias2[None, :, None, None]
    out = jax.nn.gelu(_bn(h2, g2, b2), approximate=False)
    out = pl.pallas_call(
        _copy_kernel,
        grid=(B, N // 128),
        in_specs=[pl.BlockSpec((1, 64, 128, 20), lambda b, i: (b, 0, i, 0))],
        out_specs=pl.BlockSpec((1, 64, 128, 20), lambda b, i: (b, 0, i, 0)),
        out_shape=jax.ShapeDtypeStruct(out.shape, out.dtype),
    )(out)
    return (out, xyz_t)


# speedup vs baseline: 1.1190x; 1.1190x over previous
import functools

import jax
import jax.numpy as jnp
import numpy as np
from jax import lax
from jax.experimental import pallas as pl
from jax.experimental.pallas import tpu as pltpu

K_NN = 20
PT = 2560
RT = PT // K_NN


def _seg_matrices():
    p = np.arange(PT)
    r = np.arange(RT)
    ind = (p[:, None] // K_NN == r[None, :]).astype(np.float32)
    M = ind.copy()
    E = ind.T.copy()
    return jnp.asarray(M), jnp.asarray(E)


def _dot(a, b):
    return jax.lax.dot_general(a, b, (((1,), (0,)), ((), ())),
                               precision=jax.lax.Precision.HIGHEST,
                               preferred_element_type=jnp.float32)


def _pdx_tile(f, m_ref, e_ref):
    M = m_ref[...]
    E = e_ref[...]
    inv_k = np.float32(1.0 / K_NN)
    g = f[0:3]
    q = f[3:6]
    nx = g - q
    mean_nx = _dot(_dot(nx, M) * inv_k, E)
    s_nx = jnp.sum(nx * nx, axis=0, keepdims=True)
    pdist1 = jnp.sqrt(s_nx + 1e-12)
    dxm = nx - mean_nx
    pdist2 = jnp.sqrt(jnp.sum(dxm * dxm, axis=0, keepdims=True) + 1e-12)
    norm_meanx = jnp.sqrt(jnp.sum(mean_nx * mean_nx, axis=0, keepdims=True) + 1e-12)
    cos1 = (jnp.sum(nx * mean_nx, axis=0, keepdims=True)
            / jnp.maximum(pdist1 * norm_meanx, 1e-8))
    mean_g = _dot(_dot(g, M) * inv_k, E)
    norms = g - mean_g
    nnorms = q - mean_g
    pdist3 = jnp.sqrt(4.0 * s_nx + 1e-12)
    norm_norms = jnp.sqrt(jnp.sum(norms * norms, axis=0, keepdims=True) + 1e-12)
    norm_nnorms = jnp.sqrt(jnp.sum(nnorms * nnorms, axis=0, keepdims=True) + 1e-12)
    cos2 = (jnp.sum(nx * norms, axis=0, keepdims=True)
            / jnp.maximum(pdist1 * norm_norms, 1e-8))
    cos3 = (jnp.sum(norms * nnorms, axis=0, keepdims=True)
            / jnp.maximum(norm_norms * norm_nnorms, 1e-8))
    out6 = jnp.concatenate([pdist1, pdist2, pdist3, cos1, cos2, cos3], axis=0)
    mean6 = _dot(_dot(out6, M) * inv_k, E)
    return jnp.concatenate([out6, out6 - mean6], axis=0)


def _gelu(x):
    return 0.5 * x * (1.0 + lax.erf(x * np.float32(1.0 / np.sqrt(2.0))))


def _p1_kernel(f_ref, m_ref, e_ref, s_ref):
    b = pl.program_id(0)
    i = pl.program_id(1)
    pdx = _pdx_tile(f_ref[0], m_ref, e_ref)
    ones = jnp.ones((1, PT), jnp.float32)
    aug = jnp.concatenate([pdx, ones], axis=0)
    contrib = _dot(aug, aug.T)

    @pl.when(jnp.logical_and(b == 0, i == 0))
    def _():
        s_ref[...] = jnp.zeros_like(s_ref)

    s_ref[...] += contrib


def _p2_kernel(f_ref, m_ref, e_ref, a1_ref, c1_ref, s_ref):
    b = pl.program_id(0)
    i = pl.program_id(1)
    pdx = _pdx_tile(f_ref[0], m_ref, e_ref)
    G = _gelu(_dot(a1_ref[...], pdx) + c1_ref[...])
    ones = jnp.ones((1, PT), jnp.float32)
    aug = jnp.concatenate([G, ones], axis=0)
    contrib = _dot(aug, aug.T)

    @pl.when(jnp.logical_and(b == 0, i == 0))
    def _():
        s_ref[...] = jnp.zeros_like(s_ref)

    s_ref[...] += contrib


def _p3_kernel(f_ref, m_ref, e_ref, a1_ref, c1_ref, a2_ref, c2_ref, o_ref):
    pdx = _pdx_tile(f_ref[0], m_ref, e_ref)
    G = _gelu(_dot(a1_ref[...], pdx) + c1_ref[...])
    h2 = _dot(a2_ref[...], G) + c2_ref[...]
    o_ref[0] = _gelu(h2)


def _knn_gather_scaffold(xyz):
    B, _, N = xyz.shape
    inner = -2.0 * jnp.einsum('bcn,bcm->bnm', xyz, xyz)
    xx = jnp.sum(xyz ** 2, axis=1, keepdims=True)
    pairwise = -xx - inner - jnp.transpose(xx, (0, 2, 1))
    _, idx = jax.lax.top_k(pairwise, K_NN)
    xyz_t = jnp.transpose(xyz, (0, 2, 1))
    bidx = jnp.arange(B)[:, None, None]
    gknn = xyz_t[bidx, idx]
    g = jnp.transpose(gknn, (0, 3, 1, 2)).reshape(B, 3, N * K_NN)
    q = jnp.repeat(xyz_t, K_NN, axis=1)
    q = jnp.transpose(q, (0, 2, 1))
    return jnp.concatenate([g, q], axis=1)


def kernel(x, xyz, W1, g1, b1, W2, bias2, g2, b2):
    B, _, N = xyz.shape
    P = N * K_NN
    ntiles = P // PT
    F = _knn_gather_scaffold(xyz)
    M, E = _seg_matrices()
    cnt = jnp.float32(B * P)

    f_spec = pl.BlockSpec((1, 6, PT), lambda b, i: (b, 0, i))
    m_spec = pl.BlockSpec((PT, RT), lambda b, i: (0, 0))
    e_spec = pl.BlockSpec((RT, PT), lambda b, i: (0, 0))

    S = pl.pallas_call(
        _p1_kernel,
        grid=(B, ntiles),
        in_specs=[f_spec, m_spec, e_spec],
        out_specs=pl.BlockSpec((13, 13), lambda b, i: (0, 0)),
        out_shape=jax.ShapeDtypeStruct((13, 13), jnp.float32),
    )(F, M, E)

    hp = jax.lax.Precision.HIGHEST
    s1 = S[:12, 12] / cnt
    S2n = S[:12, :12] / cnt
    mean_h = jnp.matmul(W1, s1, precision=hp)
    e2 = jnp.sum(jnp.matmul(W1, S2n, precision=hp) * W1, axis=1)
    var1 = e2 - mean_h * mean_h
    a1 = g1 / jnp.sqrt(var1 + 1e-5)
    A1 = a1[:, None] * W1
    c1 = (b1 - a1 * mean_h)[:, None]

    a1_spec = pl.BlockSpec((64, 12), lambda b, i: (0, 0))
    c1_spec = pl.BlockSpec((64, 1), lambda b, i: (0, 0))

    SG = pl.pallas_call(
        _p2_kernel,
        grid=(B, ntiles),
        in_specs=[f_spec, m_spec, e_spec, a1_spec, c1_spec],
        out_specs=pl.BlockSpec((65, 65), lambda b, i: (0, 0)),
        out_shape=jax.ShapeDtypeStruct((65, 65), jnp.float32),
    )(F, M, E, A1, c1)

    sg1 = SG[:64, 64] / cnt
    SG2n = SG[:64, :64] / cnt
    w2sg1 = jnp.matmul(W2, sg1, precision=hp)
    mean_h2 = w2sg1 + bias2
    e2b = (jnp.sum(jnp.matmul(W2, SG2n, precision=hp) * W2, axis=1)
           + 2.0 * bias2 * w2sg1 + bias2 * bias2)
    var2 = e2b - mean_h2 * mean_h2
    a2 = g2 / jnp.sqrt(var2 + 1e-5)
    A2 = a2[:, None] * W2
    c2 = (b2 + a2 * (bias2 - mean_h2))[:, None]

    a2_spec = pl.BlockSpec((64, 64), lambda b, i: (0, 0))
    c2_spec = pl.BlockSpec((64, 1), lambda b, i: (0, 0))

    out = pl.pallas_call(
        _p3_kernel,
        grid=(B, ntiles),
        in_specs=[f_spec, m_spec, e_spec, a1_spec, c1_spec, a2_spec, c2_spec],
        out_specs=pl.BlockSpec((1, 64, PT), lambda b, i: (b, 0, i)),
        out_shape=jax.ShapeDtypeStruct((B, 64, P), jnp.float32),
    )(F, M, E, A1, c1, A2, c2)

    out = out.reshape(B, 64, N, K_NN)
    xyz_t = jnp.transpose(xyz, (0, 2, 1))
    return (out, xyz_t)


# speedup vs baseline: 5.2798x; 4.7184x over previous
import functools

import jax
import jax.numpy as jnp
import numpy as np
from jax import lax
from jax.experimental import pallas as pl
from jax.experimental.pallas import tpu as pltpu
from jax.experimental.pallas import tpu_sc as plsc

K_NN = 20
PT = 2560
RT = PT // K_NN
TN = 256
NEG_INF = np.float32(-np.inf)


def _pw_kernel(xyz_ref, p_ref):
    i = pl.program_id(1)
    xyzb = xyz_ref[0]
    xx = jnp.sum(xyzb * xyzb, axis=0, keepdims=True)
    xq = xyz_ref[0, :, pl.ds(i * TN, TN)]
    inner = -2.0 * jax.lax.dot_general(
        xq, xyzb, (((0,), (0,)), ((), ())),
        preferred_element_type=jnp.float32)
    xxq = jnp.sum(xq * xq, axis=0, keepdims=True)
    p_ref[0] = (-xx) - inner - jnp.transpose(xxq)


def _pairwise(xyz):
    B, _, N = xyz.shape
    return pl.pallas_call(
        _pw_kernel,
        grid=(B, N // TN),
        in_specs=[pl.BlockSpec((1, 3, N), lambda b, i: (b, 0, 0))],
        out_specs=pl.BlockSpec((1, TN, N), lambda b, i: (b, i, 0)),
        out_shape=jax.ShapeDtypeStruct((B, N, N), jnp.float32),
    )(xyz)


SC_NC = 2
SC_NS = 16
SC_NW = SC_NC * SC_NS


def _sc_topk_gather(P, xyz):
    B, N, _ = P.shape
    nrows = (B * N) // SC_NW
    wpb = N // nrows
    mesh = plsc.VectorSubcoreMesh(core_axis_name="c", subcore_axis_name="s")

    def body(p_hbm, xyz_hbm, g_hbm, xb, yb, zb, prow, candk, candi,
             obx, oby, obz):
        i16 = lax.iota(jnp.int32, 16)
        w = lax.axis_index("s") * SC_NC + lax.axis_index("c")
        b = w // wpb
        r0 = (w % wpb) * nrows
        pltpu.sync_copy(xyz_hbm.at[b * 3 + 0, 0], xb)
        pltpu.sync_copy(xyz_hbm.at[b * 3 + 1, 0], yb)
        pltpu.sync_copy(xyz_hbm.at[b * 3 + 2, 0], zb)

        def row_body(i, _):
            n = r0 + i
            pltpu.sync_copy(p_hbm.at[b * N + n, 0], prow)

            def chunk_a(c, vm):
                vM1, vM2 = vm
                d = prow[pl.ds(c * 16, 16)]
                nm1 = jnp.maximum(vM1, d)
                nm2 = jnp.maximum(vM2, jnp.minimum(vM1, d))
                return (nm1, nm2)

            vM1, vM2 = lax.fori_loop(
                0, 256, chunk_a,
                (jnp.full((16,), NEG_INF), jnp.full((16,), NEG_INF)),
                unroll=4)

            sk1, _u1 = plsc.sort_key_val(vM1, i16, descending=True)
            sk2, _u2 = plsc.sort_key_val(vM2, i16, descending=True)
            low16 = jnp.minimum(sk1, lax.rev(sk2, (0,)))
            lows, _u3 = plsc.sort_key_val(low16, i16, descending=True)
            tv = jnp.full((16,), lows[3])

            def chunk_c(c, cur):
                d = prow[pl.ds(c * 16, 16)]
                msk = d >= tv
                plsc.store_compressed(candk.at[pl.ds(cur, 16)], d, mask=msk)
                idxv = i16 + c * 16
                plsc.store_compressed(candi.at[pl.ds(cur, 16)], idxv, mask=msk)
                return cur + plsc.all_reduce_population_count(msk)[0]

            m = lax.fori_loop(0, 256, chunk_c, 0, unroll=4)

            def merge_body(j, T):
                T0k, T0v, T1k, T1v = T
                ck = candk[pl.ds(j * 16, 16)]
                cv = candi[pl.ds(j * 16, 16)]
                ck = jnp.where(i16 < (m - j * 16), ck, NEG_INF)
                cks, cvs = plsc.sort_key_val(ck, cv, descending=True)
                rk = lax.rev(cks, (0,))
                rv = lax.rev(cvs, (0,))
                ge = T1k >= rk
                xk = jnp.where(ge, T1k, rk)
                xv = jnp.where(ge, T1v, rv)
                xks, xvs = plsc.sort_key_val(xk, xv, descending=True)
                rxk = lax.rev(xks, (0,))
                rxv = lax.rev(xvs, (0,))
                ge0 = T0k >= rxk
                n0k = jnp.where(ge0, T0k, rxk)
                n0v = jnp.where(ge0, T0v, rxv)
                n1k = jnp.where(ge0, rxk, T0k)
                n1v = jnp.where(ge0, rxv, T0v)
                n0k, n0v = plsc.sort_key_val(n0k, n0v, descending=True)
                n1k, n1v = plsc.sort_key_val(n1k, n1v, descending=True)
                return (n0k, n0v, n1k, n1v)

            zi = jnp.zeros((16,), jnp.int32)
            ninf = jnp.full((16,), NEG_INF)
            T0k, T0v, T1k, T1v = lax.fori_loop(
                0, (m + 15) // 16, merge_body, (ninf, zi, ninf, zi))

            off = (i % 32) * K_NN
            first4 = i16 < (K_NN - 16)
            for cb, ob in ((xb, obx), (yb, oby), (zb, obz)):
                g0 = plsc.load_gather(cb, [T0v])
                ob[pl.ds(off, 16)] = g0
                g1 = plsc.load_gather(cb, [T1v])
                plsc.store_compressed(
                    ob.at[pl.ds(off + 16, 16)], g1, mask=first4)

            @pl.when(i % 32 == 31)
            def _():
                base = pl.multiple_of((n - 31) * K_NN, 32 * K_NN)
                for crow, ob in ((0, obx), (1, oby), (2, obz)):
                    pltpu.sync_copy(
                        ob, g_hbm.at[b * 3 + crow, 0, pl.ds(base, 32 * K_NN)])

            return 0

        lax.fori_loop(0, nrows, row_body, 0)

    run = pl.kernel(
        body,
        out_type=jax.ShapeDtypeStruct((B * 3, 1, N * K_NN), jnp.float32),
        mesh=mesh,
        compiler_params=pltpu.CompilerParams(needs_layout_passes=False),
        scratch_types=[
            pltpu.VMEM((N,), jnp.float32),
            pltpu.VMEM((N,), jnp.float32),
            pltpu.VMEM((N,), jnp.float32),
            pltpu.VMEM((N,), jnp.float32),
            pltpu.VMEM((N + 16,), jnp.float32),
            pltpu.VMEM((N + 16,), jnp.int32),
            pltpu.VMEM((32 * K_NN,), jnp.float32),
            pltpu.VMEM((32 * K_NN,), jnp.float32),
            pltpu.VMEM((32 * K_NN,), jnp.float32),
        ],
    )
    g = run(P.reshape(B * N, 1, N), xyz.reshape(B * 3, 1, N))
    return g.reshape(B, 3, N * K_NN)


def _seg_matrices():
    p = np.arange(PT)
    r = np.arange(RT)
    ind = (p[:, None] // K_NN == r[None, :]).astype(np.float32)
    return jnp.asarray(ind), jnp.asarray(ind.T.copy())


def _dot(a, b):
    return jax.lax.dot_general(a, b, (((1,), (0,)), ((), ())),
                               precision=jax.lax.Precision.HIGHEST,
                               preferred_element_type=jnp.float32)


def _pdx_tile(f, m_ref, e_ref):
    M = m_ref[...]
    E = e_ref[...]
    inv_k = np.float32(1.0 / K_NN)
    g = f[0:3]
    q = f[3:6]
    nx = g - q
    mean_nx = _dot(_dot(nx, M) * inv_k, E)
    s_nx = jnp.sum(nx * nx, axis=0, keepdims=True)
    pdist1 = jnp.sqrt(s_nx + 1e-12)
    dxm = nx - mean_nx
    pdist2 = jnp.sqrt(jnp.sum(dxm * dxm, axis=0, keepdims=True) + 1e-12)
    norm_meanx = jnp.sqrt(jnp.sum(mean_nx * mean_nx, axis=0, keepdims=True) + 1e-12)
    cos1 = (jnp.sum(nx * mean_nx, axis=0, keepdims=True)
            / jnp.maximum(pdist1 * norm_meanx, 1e-8))
    mean_g = _dot(_dot(g, M) * inv_k, E)
    norms = g - mean_g
    nnorms = q - mean_g
    pdist3 = jnp.sqrt(4.0 * s_nx + 1e-12)
    norm_norms = jnp.sqrt(jnp.sum(norms * norms, axis=0, keepdims=True) + 1e-12)
    norm_nnorms = jnp.sqrt(jnp.sum(nnorms * nnorms, axis=0, keepdims=True) + 1e-12)
    cos2 = (jnp.sum(nx * norms, axis=0, keepdims=True)
            / jnp.maximum(pdist1 * norm_norms, 1e-8))
    cos3 = (jnp.sum(norms * nnorms, axis=0, keepdims=True)
            / jnp.maximum(norm_norms * norm_nnorms, 1e-8))
    out6 = jnp.concatenate([pdist1, pdist2, pdist3, cos1, cos2, cos3], axis=0)
    mean6 = _dot(_dot(out6, M) * inv_k, E)
    return jnp.concatenate([out6, out6 - mean6], axis=0)


def _gelu(x):
    return 0.5 * x * (1.0 + lax.erf(x * np.float32(1.0 / np.sqrt(2.0))))


def _p1_kernel(f_ref, m_ref, e_ref, s_ref):
    b = pl.program_id(0)
    i = pl.program_id(1)
    pdx = _pdx_tile(f_ref[0], m_ref, e_ref)
    ones = jnp.ones((1, PT), jnp.float32)
    aug = jnp.concatenate([pdx, ones], axis=0)
    contrib = _dot(aug, aug.T)

    @pl.when(jnp.logical_and(b == 0, i == 0))
    def _():
        s_ref[...] = jnp.zeros_like(s_ref)

    s_ref[...] += contrib


def _p2_kernel(f_ref, m_ref, e_ref, a1_ref, c1_ref, s_ref):
    b = pl.program_id(0)
    i = pl.program_id(1)
    pdx = _pdx_tile(f_ref[0], m_ref, e_ref)
    G = _gelu(_dot(a1_ref[...], pdx) + c1_ref[...])
    ones = jnp.ones((1, PT), jnp.float32)
    aug = jnp.concatenate([G, ones], axis=0)
    contrib = _dot(aug, aug.T)

    @pl.when(jnp.logical_and(b == 0, i == 0))
    def _():
        s_ref[...] = jnp.zeros_like(s_ref)

    s_ref[...] += contrib


def _p3_kernel(f_ref, m_ref, e_ref, a1_ref, c1_ref, a2_ref, c2_ref, o_ref):
    pdx = _pdx_tile(f_ref[0], m_ref, e_ref)
    G = _gelu(_dot(a1_ref[...], pdx) + c1_ref[...])
    h2 = _dot(a2_ref[...], G) + c2_ref[...]
    o_ref[0] = _gelu(h2)


def kernel(x, xyz, W1, g1, b1, W2, bias2, g2, b2):
    B, _, N = xyz.shape
    P = N * K_NN
    ntiles = P // PT

    pw = _pairwise(xyz)
    g = _sc_topk_gather(pw, xyz)
    xyz_t = jnp.transpose(xyz, (0, 2, 1))
    q = jnp.transpose(jnp.repeat(xyz_t, K_NN, axis=1), (0, 2, 1))
    F = jnp.concatenate([g, q], axis=1)

    M, E = _seg_matrices()
    cnt = jnp.float32(B * P)

    f_spec = pl.BlockSpec((1, 6, PT), lambda b, i: (b, 0, i))
    m_spec = pl.BlockSpec((PT, RT), lambda b, i: (0, 0))
    e_spec = pl.BlockSpec((RT, PT), lambda b, i: (0, 0))

    S = pl.pallas_call(
        _p1_kernel,
        grid=(B, ntiles),
        in_specs=[f_spec, m_spec, e_spec],
        out_specs=pl.BlockSpec((13, 13), lambda b, i: (0, 0)),
        out_shape=jax.ShapeDtypeStruct((13, 13), jnp.float32),
    )(F, M, E)

    hp = jax.lax.Precision.HIGHEST
    s1 = S[:12, 12] / cnt
    S2n = S[:12, :12] / cnt
    mean_h = jnp.matmul(W1, s1, precision=hp)
    e2 = jnp.sum(jnp.matmul(W1, S2n, precision=hp) * W1, axis=1)
    var1 = e2 - mean_h * mean_h
    a1 = g1 / jnp.sqrt(var1 + 1e-5)
    A1 = a1[:, None] * W1
    c1 = (b1 - a1 * mean_h)[:, None]

    a1_spec = pl.BlockSpec((64, 12), lambda b, i: (0, 0))
    c1_spec = pl.BlockSpec((64, 1), lambda b, i: (0, 0))

    SG = pl.pallas_call(
        _p2_kernel,
        grid=(B, ntiles),
        in_specs=[f_spec, m_spec, e_spec, a1_spec, c1_spec],
        out_specs=pl.BlockSpec((65, 65), lambda b, i: (0, 0)),
        out_shape=jax.ShapeDtypeStruct((65, 65), jnp.float32),
    )(F, M, E, A1, c1)

    sg1 = SG[:64, 64] / cnt
    SG2n = SG[:64, :64] / cnt
    w2sg1 = jnp.matmul(W2, sg1, precision=hp)
    mean_h2 = w2sg1 + bias2
    e2b = (jnp.sum(jnp.matmul(W2, SG2n, precision=hp) * W2, axis=1)
           + 2.0 * bias2 * w2sg1 + bias2 * bias2)
    var2 = e2b - mean_h2 * mean_h2
    a2 = g2 / jnp.sqrt(var2 + 1e-5)
    A2 = a2[:, None] * W2
    c2 = (b2 + a2 * (bias2 - mean_h2))[:, None]

    a2_spec = pl.BlockSpec((64, 64), lambda b, i: (0, 0))
    c2_spec = pl.BlockSpec((64, 1), lambda b, i: (0, 0))

    out = pl.pallas_call(
        _p3_kernel,
        grid=(B, ntiles),
        in_specs=[f_spec, m_spec, e_spec, a1_spec, c1_spec, a2_spec, c2_spec],
        out_specs=pl.BlockSpec((1, 64, PT), lambda b, i: (b, 0, i)),
        out_shape=jax.ShapeDtypeStruct((B, 64, P), jnp.float32),
    )(F, M, E, A1, c1, A2, c2)

    out = out.reshape(B, 64, N, K_NN)
    return (out, xyz_t)


# speedup vs baseline: 7.4627x; 1.4134x over previous
import functools

import jax
import jax.numpy as jnp
import numpy as np
from jax import lax
from jax.experimental import pallas as pl
from jax.experimental.pallas import tpu as pltpu
from jax.experimental.pallas import tpu_sc as plsc

K_NN = 20
PT = 2560
RT = PT // K_NN
TN = 256
NEG_INF = np.float32(-np.inf)


def _pw_kernel(xyz_ref, p_ref):
    i = pl.program_id(1)
    xyzb = xyz_ref[0]
    xx = jnp.sum(xyzb * xyzb, axis=0, keepdims=True)
    xq = xyz_ref[0, :, pl.ds(i * TN, TN)]
    inner = -2.0 * jax.lax.dot_general(
        xq, xyzb, (((0,), (0,)), ((), ())),
        preferred_element_type=jnp.float32)
    xxq = jnp.sum(xq * xq, axis=0, keepdims=True)
    p_ref[0] = (-xx) - inner - jnp.transpose(xxq)


def _pairwise(xyz):
    B, _, N = xyz.shape
    return pl.pallas_call(
        _pw_kernel,
        grid=(B, N // TN),
        in_specs=[pl.BlockSpec((1, 3, N), lambda b, i: (b, 0, 0))],
        out_specs=pl.BlockSpec((1, TN, N), lambda b, i: (b, i, 0)),
        out_shape=jax.ShapeDtypeStruct((B, N, N), jnp.float32),
    )(xyz)


SC_NC = 2
SC_NS = 16
SC_NW = SC_NC * SC_NS


def _sc_topk_gather(P, xyz):
    B, N, _ = P.shape
    nrows = (B * N) // SC_NW
    wpb = N // nrows
    mesh = plsc.VectorSubcoreMesh(core_axis_name="c", subcore_axis_name="s")

    def body(p_hbm, xyz_hbm, g_hbm, xb, yb, zb, prow, prow2, candk, candi,
             obx, oby, obz, sem0, sem1):
        w = lax.axis_index("s") * SC_NC + lax.axis_index("c")
        b = w // wpb
        r0 = (w % wpb) * nrows
        pltpu.sync_copy(xyz_hbm.at[b * 3 + 0, 0], xb)
        pltpu.sync_copy(xyz_hbm.at[b * 3 + 1, 0], yb)
        pltpu.sync_copy(xyz_hbm.at[b * 3 + 2, 0], zb)

        def process_row(prow, i, n):
            def chunk_a(j, vm):
                vm = list(vm)
                for t in range(4):
                    d = prow[pl.ds((j * 4 + t) * 16, 16)]
                    a, bb = vm[2 * t], vm[2 * t + 1]
                    na = jnp.maximum(a, d)
                    nb = jnp.maximum(bb, jnp.minimum(a, d))
                    vm[2 * t], vm[2 * t + 1] = na, nb
                return tuple(vm)

            ninf16 = jnp.full((16,), NEG_INF)
            acc = lax.fori_loop(0, 64, chunk_a, (ninf16,) * 8, unroll=2)

            def top2_merge(a1, b1, a2, b2):
                return (jnp.maximum(a1, a2),
                        jnp.maximum(jnp.minimum(a1, a2),
                                    jnp.maximum(b1, b2)))

            m1a, m2a = top2_merge(acc[0], acc[1], acc[2], acc[3])
            m1b, m2b = top2_merge(acc[4], acc[5], acc[6], acc[7])
            vM1, vM2 = top2_merge(m1a, m2a, m1b, m2b)
            i16 = lax.iota(jnp.int32, 16)

            sk1, _u1 = plsc.sort_key_val(vM1, i16, descending=True)
            sk2, _u2 = plsc.sort_key_val(vM2, i16, descending=True)
            low16 = jnp.minimum(sk1, lax.rev(sk2, (0,)))
            lows, _u3 = plsc.sort_key_val(low16, i16, descending=True)
            tv = jnp.full((16,), lows[3])

            def chunk_c(c, cur):
                d = prow[pl.ds(c * 16, 16)]
                msk = d >= tv
                plsc.store_compressed(candk.at[pl.ds(cur, 16)], d, mask=msk)
                idxv = i16 + c * 16
                plsc.store_compressed(candi.at[pl.ds(cur, 16)], idxv, mask=msk)
                return cur + plsc.all_reduce_population_count(msk)[0]

            m = lax.fori_loop(0, 256, chunk_c, 0, unroll=8)

            def merge_body(j, T):
                T0k, T0v, T1k, T1v = T
                ck = candk[pl.ds(j * 16, 16)]
                cv = candi[pl.ds(j * 16, 16)]
                ck = jnp.where(i16 < (m - j * 16), ck, NEG_INF)
                cks, cvs = plsc.sort_key_val(ck, cv, descending=True)
                rk = lax.rev(cks, (0,))
                rv = lax.rev(cvs, (0,))
                ge = T1k >= rk
                xk = jnp.where(ge, T1k, rk)
                xv = jnp.where(ge, T1v, rv)
                xks, xvs = plsc.sort_key_val(xk, xv, descending=True)
                rxk = lax.rev(xks, (0,))
                rxv = lax.rev(xvs, (0,))
                ge0 = T0k >= rxk
                n0k = jnp.where(ge0, T0k, rxk)
                n0v = jnp.where(ge0, T0v, rxv)
                n1k = jnp.where(ge0, rxk, T0k)
                n1v = jnp.where(ge0, rxv, T0v)
                n0k, n0v = plsc.sort_key_val(n0k, n0v, descending=True)
                n1k, n1v = plsc.sort_key_val(n1k, n1v, descending=True)
                return (n0k, n0v, n1k, n1v)

            zi = jnp.zeros((16,), jnp.int32)
            ninf = jnp.full((16,), NEG_INF)
            T0k, T0v, T1k, T1v = lax.fori_loop(
                0, (m + 15) // 16, merge_body, (ninf, zi, ninf, zi))

            off = (i % 32) * K_NN
            first4 = i16 < (K_NN - 16)
            for cb, ob in ((xb, obx), (yb, oby), (zb, obz)):
                g0 = plsc.load_gather(cb, [T0v])
                ob[pl.ds(off, 16)] = g0
                g1 = plsc.load_gather(cb, [T1v])
                plsc.store_compressed(
                    ob.at[pl.ds(off + 16, 16)], g1, mask=first4)

            @pl.when(i % 32 == 31)
            def _():
                base = pl.multiple_of((n - 31) * K_NN, 32 * K_NN)
                for crow, ob in ((0, obx), (1, oby), (2, obz)):
                    pltpu.sync_copy(
                        ob, g_hbm.at[b * 3 + crow, 0, pl.ds(base, 32 * K_NN)])

        last = r0 + nrows - 1
        cp0 = pltpu.async_copy(p_hbm.at[b * N + r0, 0], prow, sem0)
        cp0.wait()

        def pair_body(i2, _):
            i0 = 2 * i2
            n0 = r0 + i0
            pltpu.async_copy(
                p_hbm.at[b * N + jnp.minimum(n0 + 1, last), 0], prow2, sem1)
            process_row(prow, i0, n0)
            pltpu.make_async_copy(
                p_hbm.at[b * N + n0, 0], prow2, sem1).wait()
            pltpu.async_copy(
                p_hbm.at[b * N + jnp.minimum(n0 + 2, last), 0], prow, sem0)
            process_row(prow2, i0 + 1, n0 + 1)
            pltpu.make_async_copy(
                p_hbm.at[b * N + n0, 0], prow, sem0).wait()
            return 0

        lax.fori_loop(0, nrows // 2, pair_body, 0)

    run = pl.kernel(
        body,
        out_type=jax.ShapeDtypeStruct((B * 3, 1, N * K_NN), jnp.float32),
        mesh=mesh,
        compiler_params=pltpu.CompilerParams(needs_layout_passes=False),
        scratch_types=[
            pltpu.VMEM((N,), jnp.float32),
            pltpu.VMEM((N,), jnp.float32),
            pltpu.VMEM((N,), jnp.float32),
            pltpu.VMEM((N,), jnp.float32),
            pltpu.VMEM((N,), jnp.float32),
            pltpu.VMEM((N + 16,), jnp.float32),
            pltpu.VMEM((N + 16,), jnp.int32),
            pltpu.VMEM((32 * K_NN,), jnp.float32),
            pltpu.VMEM((32 * K_NN,), jnp.float32),
            pltpu.VMEM((32 * K_NN,), jnp.float32),
            pltpu.SemaphoreType.DMA,
            pltpu.SemaphoreType.DMA,
        ],
    )
    g = run(P.reshape(B * N, 1, N), xyz.reshape(B * 3, 1, N))
    return g.reshape(B, 3, N * K_NN)


def _seg_matrices():
    p = np.arange(PT)
    r = np.arange(RT)
    ind = (p[:, None] // K_NN == r[None, :]).astype(np.float32)
    return jnp.asarray(ind), jnp.asarray(ind.T.copy())


def _dot(a, b):
    return jax.lax.dot_general(a, b, (((1,), (0,)), ((), ())),
                               precision=jax.lax.Precision.HIGHEST,
                               preferred_element_type=jnp.float32)


def _dot_lp(a, b):
    return jax.lax.dot_general(a, b, (((1,), (0,)), ((), ())),
                               preferred_element_type=jnp.float32)


def _pdx_tile(f, m_ref, e_ref):
    M = m_ref[...]
    E = e_ref[...]
    inv_k = np.float32(1.0 / K_NN)
    g = f[0:3]
    q = f[3:6]
    nx = g - q
    mean_nx = _dot(_dot(nx, M) * inv_k, E)
    s_nx = jnp.sum(nx * nx, axis=0, keepdims=True)
    pdist1 = jnp.sqrt(s_nx + 1e-12)
    dxm = nx - mean_nx
    pdist2 = jnp.sqrt(jnp.sum(dxm * dxm, axis=0, keepdims=True) + 1e-12)
    norm_meanx = jnp.sqrt(jnp.sum(mean_nx * mean_nx, axis=0, keepdims=True) + 1e-12)
    cos1 = (jnp.sum(nx * mean_nx, axis=0, keepdims=True)
            / jnp.maximum(pdist1 * norm_meanx, 1e-8))
    mean_g = _dot(_dot(g, M) * inv_k, E)
    norms = g - mean_g
    nnorms = q - mean_g
    pdist3 = jnp.sqrt(4.0 * s_nx + 1e-12)
    norm_norms = jnp.sqrt(jnp.sum(norms * norms, axis=0, keepdims=True) + 1e-12)
    norm_nnorms = jnp.sqrt(jnp.sum(nnorms * nnorms, axis=0, keepdims=True) + 1e-12)
    cos2 = (jnp.sum(nx * norms, axis=0, keepdims=True)
            / jnp.maximum(pdist1 * norm_norms, 1e-8))
    cos3 = (jnp.sum(norms * nnorms, axis=0, keepdims=True)
            / jnp.maximum(norm_norms * norm_nnorms, 1e-8))
    out6 = jnp.concatenate([pdist1, pdist2, pdist3, cos1, cos2, cos3], axis=0)
    mean6 = _dot(_dot(out6, M) * inv_k, E)
    return jnp.concatenate([out6, out6 - mean6], axis=0)


def _gelu(x):
    return 0.5 * x * (1.0 + lax.erf(x * np.float32(1.0 / np.sqrt(2.0))))


def _p1_kernel(f_ref, m_ref, e_ref, s_ref, pdx_ref):
    b = pl.program_id(0)
    i = pl.program_id(1)
    pdx = _pdx_tile(f_ref[0], m_ref, e_ref)
    pdx_ref[0] = pdx
    ones = jnp.ones((1, PT), jnp.float32)
    aug = jnp.concatenate([pdx, ones], axis=0)
    contrib = _dot_lp(aug, aug.T)

    @pl.when(jnp.logical_and(b == 0, i == 0))
    def _():
        s_ref[...] = jnp.zeros_like(s_ref)

    s_ref[...] += contrib


def _p2_kernel(pdx_ref, a1_ref, c1_ref, s_ref):
    b = pl.program_id(0)
    i = pl.program_id(1)
    pdx = pdx_ref[0]
    G = _gelu(_dot(a1_ref[...], pdx) + c1_ref[...])
    ones = jnp.ones((1, PT), jnp.float32)
    aug = jnp.concatenate([G, ones], axis=0)
    contrib = _dot_lp(aug, aug.T)

    @pl.when(jnp.logical_and(b == 0, i == 0))
    def _():
        s_ref[...] = jnp.zeros_like(s_ref)

    s_ref[...] += contrib


def _p3_kernel(pdx_ref, a1_ref, c1_ref, a2_ref, c2_ref, o_ref):
    pdx = pdx_ref[0]
    G = _gelu(_dot(a1_ref[...], pdx) + c1_ref[...])
    h2 = _dot(a2_ref[...], G) + c2_ref[...]
    o_ref[0] = _gelu(h2)


def kernel(x, xyz, W1, g1, b1, W2, bias2, g2, b2):
    B, _, N = xyz.shape
    P = N * K_NN
    ntiles = P // PT

    pw = _pairwise(xyz)
    g = _sc_topk_gather(pw, xyz)
    xyz_t = jnp.transpose(xyz, (0, 2, 1))
    q = jnp.transpose(jnp.repeat(xyz_t, K_NN, axis=1), (0, 2, 1))
    F = jnp.concatenate([g, q], axis=1)

    M, E = _seg_matrices()
    cnt = jnp.float32(B * P)

    f_spec = pl.BlockSpec((1, 6, PT), lambda b, i: (b, 0, i))
    m_spec = pl.BlockSpec((PT, RT), lambda b, i: (0, 0))
    e_spec = pl.BlockSpec((RT, PT), lambda b, i: (0, 0))

    pdx_spec = pl.BlockSpec((1, 12, PT), lambda b, i: (b, 0, i))
    S, PDX = pl.pallas_call(
        _p1_kernel,
        grid=(B, ntiles),
        in_specs=[f_spec, m_spec, e_spec],
        out_specs=[pl.BlockSpec((13, 13), lambda b, i: (0, 0)), pdx_spec],
        out_shape=[jax.ShapeDtypeStruct((13, 13), jnp.float32),
                   jax.ShapeDtypeStruct((B, 12, P), jnp.float32)],
    )(F, M, E)

    hp = jax.lax.Precision.HIGHEST
    s1 = S[:12, 12] / cnt
    S2n = S[:12, :12] / cnt
    mean_h = jnp.matmul(W1, s1, precision=hp)
    e2 = jnp.sum(jnp.matmul(W1, S2n, precision=hp) * W1, axis=1)
    var1 = e2 - mean_h * mean_h
    a1 = g1 / jnp.sqrt(var1 + 1e-5)
    A1 = a1[:, None] * W1
    c1 = (b1 - a1 * mean_h)[:, None]

    a1_spec = pl.BlockSpec((64, 12), lambda b, i: (0, 0))
    c1_spec = pl.BlockSpec((64, 1), lambda b, i: (0, 0))

    SG = pl.pallas_call(
        _p2_kernel,
        grid=(B, ntiles),
        in_specs=[pdx_spec, a1_spec, c1_spec],
        out_specs=pl.BlockSpec((65, 65), lambda b, i: (0, 0)),
        out_shape=jax.ShapeDtypeStruct((65, 65), jnp.float32),
    )(PDX, A1, c1)

    sg1 = SG[:64, 64] / cnt
    SG2n = SG[:64, :64] / cnt
    w2sg1 = jnp.matmul(W2, sg1, precision=hp)
    mean_h2 = w2sg1 + bias2
    e2b = (jnp.sum(jnp.matmul(W2, SG2n, precision=hp) * W2, axis=1)
           + 2.0 * bias2 * w2sg1 + bias2 * bias2)
    var2 = e2b - mean_h2 * mean_h2
    a2 = g2 / jnp.sqrt(var2 + 1e-5)
    A2 = a2[:, None] * W2
    c2 = (b2 + a2 * (bias2 - mean_h2))[:, None]

    a2_spec = pl.BlockSpec((64, 64), lambda b, i: (0, 0))
    c2_spec = pl.BlockSpec((64, 1), lambda b, i: (0, 0))

    out = pl.pallas_call(
        _p3_kernel,
        grid=(B, ntiles),
        in_specs=[pdx_spec, a1_spec, c1_spec, a2_spec, c2_spec],
        out_specs=pl.BlockSpec((1, 64, PT), lambda b, i: (b, 0, i)),
        out_shape=jax.ShapeDtypeStruct((B, 64, P), jnp.float32),
    )(PDX, A1, c1, A2, c2)

    out = out.reshape(B, 64, N, K_NN)
    return (out, xyz_t)


# speedup vs baseline: 8.3757x; 1.1223x over previous
import functools

import jax
import jax.numpy as jnp
import numpy as np
from jax import lax
from jax.experimental import pallas as pl
from jax.experimental.pallas import tpu as pltpu
from jax.experimental.pallas import tpu_sc as plsc

K_NN = 20
PT = 2560
RT = PT // K_NN
TN = 256
NEG_INF = np.float32(-np.inf)


def _pw_kernel(xyz_ref, p_ref):
    i = pl.program_id(1)
    xyzb = xyz_ref[0]
    xx = jnp.sum(xyzb * xyzb, axis=0, keepdims=True)
    xq = xyz_ref[0, :, pl.ds(i * TN, TN)]
    inner = -2.0 * jax.lax.dot_general(
        xq, xyzb, (((0,), (0,)), ((), ())),
        preferred_element_type=jnp.float32)
    xxq = jnp.sum(xq * xq, axis=0, keepdims=True)
    p_ref[0] = (-xx) - inner - jnp.transpose(xxq)


def _pairwise(xyz):
    B, _, N = xyz.shape
    return pl.pallas_call(
        _pw_kernel,
        grid=(B, N // TN),
        in_specs=[pl.BlockSpec((1, 3, N), lambda b, i: (b, 0, 0))],
        out_specs=pl.BlockSpec((1, TN, N), lambda b, i: (b, i, 0)),
        out_shape=jax.ShapeDtypeStruct((B, N, N), jnp.float32),
    )(xyz)


SC_NC = 2
SC_NS = 16
SC_NW = SC_NC * SC_NS


def _sc_topk_gather(P, xyz):
    B, N, _ = P.shape
    nrows = (B * N) // SC_NW
    wpb = N // nrows
    mesh = plsc.VectorSubcoreMesh(core_axis_name="c", subcore_axis_name="s")

    def body(p_hbm, xyz_hbm, g_hbm, xb, yb, zb, prow, prow2, candk, candi,
             candk2, candi2, obx, oby, obz, sem0, sem1):
        w = lax.axis_index("s") * SC_NC + lax.axis_index("c")
        b = w // wpb
        r0 = (w % wpb) * nrows
        pltpu.sync_copy(xyz_hbm.at[b * 3 + 0, 0], xb)
        pltpu.sync_copy(xyz_hbm.at[b * 3 + 1, 0], yb)
        pltpu.sync_copy(xyz_hbm.at[b * 3 + 2, 0], zb)

        def process_row(prow, i, n):
            def chunk_a(j, vm):
                vm = list(vm)
                for t in range(4):
                    d = prow[pl.ds((j * 4 + t) * 16, 16)]
                    a, bb = vm[2 * t], vm[2 * t + 1]
                    na = jnp.maximum(a, d)
                    nb = jnp.maximum(bb, jnp.minimum(a, d))
                    vm[2 * t], vm[2 * t + 1] = na, nb
                return tuple(vm)

            ninf16 = jnp.full((16,), NEG_INF)
            acc = lax.fori_loop(0, 64, chunk_a, (ninf16,) * 8, unroll=2)

            def top2_merge(a1, b1, a2, b2):
                return (jnp.maximum(a1, a2),
                        jnp.maximum(jnp.minimum(a1, a2),
                                    jnp.maximum(b1, b2)))

            m1a, m2a = top2_merge(acc[0], acc[1], acc[2], acc[3])
            m1b, m2b = top2_merge(acc[4], acc[5], acc[6], acc[7])
            vM1, vM2 = top2_merge(m1a, m2a, m1b, m2b)
            i16 = lax.iota(jnp.int32, 16)

            sk1, _u1 = plsc.sort_key_val(vM1, i16, descending=True)
            sk2, _u2 = plsc.sort_key_val(vM2, i16, descending=True)
            low16 = jnp.minimum(sk1, lax.rev(sk2, (0,)))
            lows, _u3 = plsc.sort_key_val(low16, i16, descending=True)
            tv = jnp.full((16,), lows[3])

            def chunk_c(j, carry):
                ca, cb = carry
                da = prow[pl.ds(j * 16, 16)]
                ma = da >= tv
                plsc.store_compressed(candk.at[pl.ds(ca, 16)], da, mask=ma)
                plsc.store_compressed(candi.at[pl.ds(ca, 16)], i16 + j * 16,
                                      mask=ma)
                db = prow[pl.ds((128 + j) * 16, 16)]
                mb = db >= tv
                plsc.store_compressed(candk2.at[pl.ds(cb, 16)], db, mask=mb)
                plsc.store_compressed(candi2.at[pl.ds(cb, 16)],
                                      i16 + (128 + j) * 16, mask=mb)
                return (ca + plsc.all_reduce_population_count(ma)[0],
                        cb + plsc.all_reduce_population_count(mb)[0])

            ma_n, mb_n = lax.fori_loop(0, 128, chunk_c, (0, 0), unroll=4)

            def make_merge(kref, iref, m):
              def merge_body(j, T):
                T0k, T0v, T1k, T1v = T
                ck = kref[pl.ds(j * 16, 16)]
                cv = iref[pl.ds(j * 16, 16)]
                ck = jnp.where(i16 < (m - j * 16), ck, NEG_INF)
                cks, cvs = plsc.sort_key_val(ck, cv, descending=True)
                rk = lax.rev(cks, (0,))
                rv = lax.rev(cvs, (0,))
                ge = T1k >= rk
                xk = jnp.where(ge, T1k, rk)
                xv = jnp.where(ge, T1v, rv)
                xks, xvs = plsc.sort_key_val(xk, xv, descending=True)
                rxk = lax.rev(xks, (0,))
                rxv = lax.rev(xvs, (0,))
                ge0 = T0k >= rxk
                n0k = jnp.where(ge0, T0k, rxk)
                n0v = jnp.where(ge0, T0v, rxv)
                n1k = jnp.where(ge0, rxk, T0k)
                n1v = jnp.where(ge0, rxv, T0v)
                n0k, n0v = plsc.sort_key_val(n0k, n0v, descending=True)
                n1k, n1v = plsc.sort_key_val(n1k, n1v, descending=True)
                return (n0k, n0v, n1k, n1v)
              return merge_body

            zi = jnp.zeros((16,), jnp.int32)
            ninf = jnp.full((16,), NEG_INF)
            T = lax.fori_loop(0, (ma_n + 15) // 16,
                              make_merge(candk, candi, ma_n),
                              (ninf, zi, ninf, zi))
            T0k, T0v, T1k, T1v = lax.fori_loop(
                0, (mb_n + 15) // 16, make_merge(candk2, candi2, mb_n), T)

            off = (i % 32) * K_NN
            first4 = i16 < (K_NN - 16)
            for cb, ob in ((xb, obx), (yb, oby), (zb, obz)):
                g0 = plsc.load_gather(cb, [T0v])
                ob[pl.ds(off, 16)] = g0
                g1 = plsc.load_gather(cb, [T1v])
                plsc.store_compressed(
                    ob.at[pl.ds(off + 16, 16)], g1, mask=first4)

            @pl.when(i % 32 == 31)
            def _():
                base = pl.multiple_of((n - 31) * K_NN, 32 * K_NN)
                for crow, ob in ((0, obx), (1, oby), (2, obz)):
                    pltpu.sync_copy(
                        ob, g_hbm.at[b * 3 + crow, 0, pl.ds(base, 32 * K_NN)])

        last = r0 + nrows - 1
        cp0 = pltpu.async_copy(p_hbm.at[b * N + r0, 0], prow, sem0)
        cp0.wait()

        def pair_body(i2, _):
            i0 = 2 * i2
            n0 = r0 + i0
            pltpu.async_copy(
                p_hbm.at[b * N + jnp.minimum(n0 + 1, last), 0], prow2, sem1)
            process_row(prow, i0, n0)
            pltpu.make_async_copy(
                p_hbm.at[b * N + n0, 0], prow2, sem1).wait()
            pltpu.async_copy(
                p_hbm.at[b * N + jnp.minimum(n0 + 2, last), 0], prow, sem0)
            process_row(prow2, i0 + 1, n0 + 1)
            pltpu.make_async_copy(
                p_hbm.at[b * N + n0, 0], prow, sem0).wait()
            return 0

        lax.fori_loop(0, nrows // 2, pair_body, 0)

    run = pl.kernel(
        body,
        out_type=jax.ShapeDtypeStruct((B * 3, 1, N * K_NN), jnp.float32),
        mesh=mesh,
        compiler_params=pltpu.CompilerParams(needs_layout_passes=False),
        scratch_types=[
            pltpu.VMEM((N,), jnp.float32),
            pltpu.VMEM((N,), jnp.float32),
            pltpu.VMEM((N,), jnp.float32),
            pltpu.VMEM((N,), jnp.float32),
            pltpu.VMEM((N,), jnp.float32),
            pltpu.VMEM((N + 16,), jnp.float32),
            pltpu.VMEM((N + 16,), jnp.int32),
            pltpu.VMEM((N + 16,), jnp.float32),
            pltpu.VMEM((N + 16,), jnp.int32),
            pltpu.VMEM((32 * K_NN,), jnp.float32),
            pltpu.VMEM((32 * K_NN,), jnp.float32),
            pltpu.VMEM((32 * K_NN,), jnp.float32),
            pltpu.SemaphoreType.DMA,
            pltpu.SemaphoreType.DMA,
        ],
    )
    g = run(P.reshape(B * N, 1, N), xyz.reshape(B * 3, 1, N))
    return g.reshape(B, 3, N * K_NN)


def _seg_matrices():
    p = np.arange(PT)
    r = np.arange(RT)
    ind = (p[:, None] // K_NN == r[None, :]).astype(np.float32)
    return jnp.asarray(ind), jnp.asarray(ind.T.copy())


def _dot(a, b):
    return jax.lax.dot_general(a, b, (((1,), (0,)), ((), ())),
                               precision=jax.lax.Precision.HIGHEST,
                               preferred_element_type=jnp.float32)


def _dot_lp(a, b):
    return jax.lax.dot_general(a, b, (((1,), (0,)), ((), ())),
                               preferred_element_type=jnp.float32)


def _pdx_tile(g, q, m_ref, e_ref):
    M = m_ref[...]
    E = e_ref[...]
    inv_k = np.float32(1.0 / K_NN)
    nx = g - q
    means = _dot(_dot(jnp.concatenate([nx, g], axis=0), M) * inv_k, E)
    mean_nx = means[0:3]
    s_nx = jnp.sum(nx * nx, axis=0, keepdims=True)
    pdist1 = jnp.sqrt(s_nx + 1e-12)
    dxm = nx - mean_nx
    pdist2 = jnp.sqrt(jnp.sum(dxm * dxm, axis=0, keepdims=True) + 1e-12)
    norm_meanx = jnp.sqrt(jnp.sum(mean_nx * mean_nx, axis=0, keepdims=True) + 1e-12)
    cos1 = (jnp.sum(nx * mean_nx, axis=0, keepdims=True)
            / jnp.maximum(pdist1 * norm_meanx, 1e-8))
    mean_g = means[3:6]
    norms = g - mean_g
    nnorms = q - mean_g
    pdist3 = jnp.sqrt(4.0 * s_nx + 1e-12)
    norm_norms = jnp.sqrt(jnp.sum(norms * norms, axis=0, keepdims=True) + 1e-12)
    norm_nnorms = jnp.sqrt(jnp.sum(nnorms * nnorms, axis=0, keepdims=True) + 1e-12)
    cos2 = (jnp.sum(nx * norms, axis=0, keepdims=True)
            / jnp.maximum(pdist1 * norm_norms, 1e-8))
    cos3 = (jnp.sum(norms * nnorms, axis=0, keepdims=True)
            / jnp.maximum(norm_norms * norm_nnorms, 1e-8))
    out6 = jnp.concatenate([pdist1, pdist2, pdist3, cos1, cos2, cos3], axis=0)
    mean6 = _dot(_dot(out6, M) * inv_k, E)
    return jnp.concatenate([out6, out6 - mean6], axis=0)


def _gelu(x):
    return 0.5 * x * (1.0 + lax.erf(x * np.float32(1.0 / np.sqrt(2.0))))


def _p1_kernel(g_ref, q_ref, m_ref, e_ref, s_ref, pdx_ref):
    b = pl.program_id(0)
    i = pl.program_id(1)
    pdx = _pdx_tile(g_ref[0], q_ref[0], m_ref, e_ref)
    pdx_ref[0] = pdx
    ones = jnp.ones((1, PT), jnp.float32)
    aug = jnp.concatenate([pdx, ones], axis=0)
    contrib = _dot_lp(aug, aug.T)

    @pl.when(jnp.logical_and(b == 0, i == 0))
    def _():
        s_ref[...] = jnp.zeros_like(s_ref)

    s_ref[...] += contrib


def _p2_kernel(pdx_ref, a1_ref, c1_ref, s_ref):
    b = pl.program_id(0)
    i = pl.program_id(1)
    pdx = pdx_ref[0]
    G = _gelu(_dot(a1_ref[...], pdx) + c1_ref[...])
    ones = jnp.ones((1, PT), jnp.float32)
    aug = jnp.concatenate([G, ones], axis=0)
    contrib = _dot_lp(aug, aug.T)

    @pl.when(jnp.logical_and(b == 0, i == 0))
    def _():
        s_ref[...] = jnp.zeros_like(s_ref)

    s_ref[...] += contrib


def _p3_kernel(pdx_ref, a1_ref, c1_ref, a2_ref, c2_ref, o_ref):
    pdx = pdx_ref[0]
    G = _gelu(_dot(a1_ref[...], pdx) + c1_ref[...])
    h2 = _dot(a2_ref[...], G) + c2_ref[...]
    o_ref[0] = _gelu(h2)


def kernel(x, xyz, W1, g1, b1, W2, bias2, g2, b2):
    B, _, N = xyz.shape
    P = N * K_NN
    ntiles = P // PT

    pw = _pairwise(xyz)
    g = _sc_topk_gather(pw, xyz)
    xyz_t = jnp.transpose(xyz, (0, 2, 1))
    q = jnp.transpose(jnp.repeat(xyz_t, K_NN, axis=1), (0, 2, 1))

    M, E = _seg_matrices()
    cnt = jnp.float32(B * P)

    f_spec = pl.BlockSpec((1, 3, PT), lambda b, i: (b, 0, i))
    m_spec = pl.BlockSpec((PT, RT), lambda b, i: (0, 0))
    e_spec = pl.BlockSpec((RT, PT), lambda b, i: (0, 0))

    pdx_spec = pl.BlockSpec((1, 12, PT), lambda b, i: (b, 0, i))
    S, PDX = pl.pallas_call(
        _p1_kernel,
        grid=(B, ntiles),
        in_specs=[f_spec, f_spec, m_spec, e_spec],
        out_specs=[pl.BlockSpec((13, 13), lambda b, i: (0, 0)), pdx_spec],
        out_shape=[jax.ShapeDtypeStruct((13, 13), jnp.float32),
                   jax.ShapeDtypeStruct((B, 12, P), jnp.float32)],
    )(g, q, M, E)

    hp = jax.lax.Precision.HIGHEST
    s1 = S[:12, 12] / cnt
    S2n = S[:12, :12] / cnt
    mean_h = jnp.matmul(W1, s1, precision=hp)
    e2 = jnp.sum(jnp.matmul(W1, S2n, precision=hp) * W1, axis=1)
    var1 = e2 - mean_h * mean_h
    a1 = g1 / jnp.sqrt(var1 + 1e-5)
    A1 = a1[:, None] * W1
    c1 = (b1 - a1 * mean_h)[:, None]

    a1_spec = pl.BlockSpec((64, 12), lambda b, i: (0, 0))
    c1_spec = pl.BlockSpec((64, 1), lambda b, i: (0, 0))

    SG = pl.pallas_call(
        _p2_kernel,
        grid=(B, ntiles),
        in_specs=[pdx_spec, a1_spec, c1_spec],
        out_specs=pl.BlockSpec((65, 65), lambda b, i: (0, 0)),
        out_shape=jax.ShapeDtypeStruct((65, 65), jnp.float32),
    )(PDX, A1, c1)

    sg1 = SG[:64, 64] / cnt
    SG2n = SG[:64, :64] / cnt
    w2sg1 = jnp.matmul(W2, sg1, precision=hp)
    mean_h2 = w2sg1 + bias2
    e2b = (jnp.sum(jnp.matmul(W2, SG2n, precision=hp) * W2, axis=1)
           + 2.0 * bias2 * w2sg1 + bias2 * bias2)
    var2 = e2b - mean_h2 * mean_h2
    a2 = g2 / jnp.sqrt(var2 + 1e-5)
    A2 = a2[:, None] * W2
    c2 = (b2 + a2 * (bias2 - mean_h2))[:, None]

    a2_spec = pl.BlockSpec((64, 64), lambda b, i: (0, 0))
    c2_spec = pl.BlockSpec((64, 1), lambda b, i: (0, 0))

    out = pl.pallas_call(
        _p3_kernel,
        grid=(B, ntiles),
        in_specs=[pdx_spec, a1_spec, c1_spec, a2_spec, c2_spec],
        out_specs=pl.BlockSpec((1, 64, PT), lambda b, i: (b, 0, i)),
        out_shape=jax.ShapeDtypeStruct((B, 64, P), jnp.float32),
    )(PDX, A1, c1, A2, c2)

    out = out.reshape(B, 64, N, K_NN)
    return (out, xyz_t)


# speedup vs baseline: 11.6088x; 1.3860x over previous
import functools

import jax
import jax.numpy as jnp
import numpy as np
from jax import lax
from jax.experimental import pallas as pl
from jax.experimental.pallas import tpu as pltpu
from jax.experimental.pallas import tpu_sc as plsc

K_NN = 20
PT = 2560
RT = PT // K_NN
TN = 256
NEG_INF = np.float32(-np.inf)


def _pw_kernel(xyz_ref, p_ref):
    i = pl.program_id(1)
    xyzb = xyz_ref[0]
    xx = jnp.sum(xyzb * xyzb, axis=0, keepdims=True)
    xq = xyz_ref[0, :, pl.ds(i * TN, TN)]
    inner = -2.0 * jax.lax.dot_general(
        xq, xyzb, (((0,), (0,)), ((), ())),
        preferred_element_type=jnp.float32)
    xxq = jnp.sum(xq * xq, axis=0, keepdims=True)
    p_ref[0] = (-xx) - inner - jnp.transpose(xxq)


def _pairwise(xyz):
    B, _, N = xyz.shape
    return pl.pallas_call(
        _pw_kernel,
        grid=(B, N // TN),
        in_specs=[pl.BlockSpec((1, 3, N), lambda b, i: (b, 0, 0))],
        out_specs=pl.BlockSpec((1, TN, N), lambda b, i: (b, i, 0)),
        out_shape=jax.ShapeDtypeStruct((B, N, N), jnp.float32),
    )(xyz)


SC_NC = 2
SC_NS = 16
SC_NW = SC_NC * SC_NS


def _sc_topk_gather(P, xyz):
    B, N, _ = P.shape
    nrows = (B * N) // SC_NW
    wpb = N // nrows
    mesh = plsc.VectorSubcoreMesh(core_axis_name="c", subcore_axis_name="s")

    def body(p_hbm, xyz_hbm, g_hbm, xb, yb, zb, prow, prow2, candk, candi,
             candk2, candi2, obx, oby, obz, sem0, sem1):
        w = lax.axis_index("s") * SC_NC + lax.axis_index("c")
        b = w // wpb
        r0 = (w % wpb) * nrows
        pltpu.sync_copy(xyz_hbm.at[b * 3 + 0, 0], xb)
        pltpu.sync_copy(xyz_hbm.at[b * 3 + 1, 0], yb)
        pltpu.sync_copy(xyz_hbm.at[b * 3 + 2, 0], zb)

        def process_row(prow, i, n):
            def chunk_a(j, vm):
                vm = list(vm)
                for t in range(4):
                    d = prow[pl.ds((j * 4 + t) * 16, 16)]
                    a, bb = vm[2 * t], vm[2 * t + 1]
                    na = jnp.maximum(a, d)
                    nb = jnp.maximum(bb, jnp.minimum(a, d))
                    vm[2 * t], vm[2 * t + 1] = na, nb
                return tuple(vm)

            ninf16 = jnp.full((16,), NEG_INF)
            acc = lax.fori_loop(0, 64, chunk_a, (ninf16,) * 8, unroll=2)

            def top2_merge(a1, b1, a2, b2):
                return (jnp.maximum(a1, a2),
                        jnp.maximum(jnp.minimum(a1, a2),
                                    jnp.maximum(b1, b2)))

            m1a, m2a = top2_merge(acc[0], acc[1], acc[2], acc[3])
            m1b, m2b = top2_merge(acc[4], acc[5], acc[6], acc[7])
            vM1, vM2 = top2_merge(m1a, m2a, m1b, m2b)
            i16 = lax.iota(jnp.int32, 16)

            sk1, _u1 = plsc.sort_key_val(vM1, i16, descending=True)
            sk2, _u2 = plsc.sort_key_val(vM2, i16, descending=True)
            low16 = jnp.minimum(sk1, lax.rev(sk2, (0,)))
            lows, _u3 = plsc.sort_key_val(low16, i16, descending=True)
            tv = jnp.full((16,), lows[3])

            def chunk_c(j, carry):
                ca, cb = carry
                da = prow[pl.ds(j * 16, 16)]
                ma = da >= tv
                plsc.store_compressed(candk.at[pl.ds(ca, 16)], da, mask=ma)
                plsc.store_compressed(candi.at[pl.ds(ca, 16)], i16 + j * 16,
                                      mask=ma)
                db = prow[pl.ds((128 + j) * 16, 16)]
                mb = db >= tv
                plsc.store_compressed(candk2.at[pl.ds(cb, 16)], db, mask=mb)
                plsc.store_compressed(candi2.at[pl.ds(cb, 16)],
                                      i16 + (128 + j) * 16, mask=mb)
                return (ca + plsc.all_reduce_population_count(ma)[0],
                        cb + plsc.all_reduce_population_count(mb)[0])

            ma_n, mb_n = lax.fori_loop(0, 128, chunk_c, (0, 0), unroll=4)

            def make_merge(kref, iref, m):
              def merge_body(j, T):
                T0k, T0v, T1k, T1v = T
                ck = kref[pl.ds(j * 16, 16)]
                cv = iref[pl.ds(j * 16, 16)]
                ck = jnp.where(i16 < (m - j * 16), ck, NEG_INF)
                cks, cvs = plsc.sort_key_val(ck, cv, descending=True)
                rk = lax.rev(cks, (0,))
                rv = lax.rev(cvs, (0,))
                ge = T1k >= rk
                xk = jnp.where(ge, T1k, rk)
                xv = jnp.where(ge, T1v, rv)
                xks, xvs = plsc.sort_key_val(xk, xv, descending=True)
                rxk = lax.rev(xks, (0,))
                rxv = lax.rev(xvs, (0,))
                ge0 = T0k >= rxk
                n0k = jnp.where(ge0, T0k, rxk)
                n0v = jnp.where(ge0, T0v, rxv)
                n1k = jnp.where(ge0, rxk, T0k)
                n1v = jnp.where(ge0, rxv, T0v)
                n0k, n0v = plsc.sort_key_val(n0k, n0v, descending=True)
                n1k, n1v = plsc.sort_key_val(n1k, n1v, descending=True)
                return (n0k, n0v, n1k, n1v)
              return merge_body

            zi = jnp.zeros((16,), jnp.int32)
            ninf = jnp.full((16,), NEG_INF)
            T = lax.fori_loop(0, (ma_n + 15) // 16,
                              make_merge(candk, candi, ma_n),
                              (ninf, zi, ninf, zi))
            T0k, T0v, T1k, T1v = lax.fori_loop(
                0, (mb_n + 15) // 16, make_merge(candk2, candi2, mb_n), T)

            off = (i % 32) * K_NN
            first4 = i16 < (K_NN - 16)
            for cb, ob in ((xb, obx), (yb, oby), (zb, obz)):
                g0 = plsc.load_gather(cb, [T0v])
                ob[pl.ds(off, 16)] = g0
                g1 = plsc.load_gather(cb, [T1v])
                plsc.store_compressed(
                    ob.at[pl.ds(off + 16, 16)], g1, mask=first4)

            @pl.when(i % 32 == 31)
            def _():
                base = pl.multiple_of((n - 31) * K_NN, 32 * K_NN)
                for crow, ob in ((0, obx), (1, oby), (2, obz)):
                    pltpu.sync_copy(
                        ob, g_hbm.at[b * 3 + crow, 0, pl.ds(base, 32 * K_NN)])

        last = r0 + nrows - 1
        cp0 = pltpu.async_copy(p_hbm.at[b * N + r0, 0], prow, sem0)
        cp0.wait()

        def pair_body(i2, _):
            i0 = 2 * i2
            n0 = r0 + i0
            pltpu.async_copy(
                p_hbm.at[b * N + jnp.minimum(n0 + 1, last), 0], prow2, sem1)
            process_row(prow, i0, n0)
            pltpu.make_async_copy(
                p_hbm.at[b * N + n0, 0], prow2, sem1).wait()
            pltpu.async_copy(
                p_hbm.at[b * N + jnp.minimum(n0 + 2, last), 0], prow, sem0)
            process_row(prow2, i0 + 1, n0 + 1)
            pltpu.make_async_copy(
                p_hbm.at[b * N + n0, 0], prow, sem0).wait()
            return 0

        lax.fori_loop(0, nrows // 2, pair_body, 0)

    run = pl.kernel(
        body,
        out_type=jax.ShapeDtypeStruct((B * 3, 1, N * K_NN), jnp.float32),
        mesh=mesh,
        compiler_params=pltpu.CompilerParams(needs_layout_passes=False),
        scratch_types=[
            pltpu.VMEM((N,), jnp.float32),
            pltpu.VMEM((N,), jnp.float32),
            pltpu.VMEM((N,), jnp.float32),
            pltpu.VMEM((N,), jnp.float32),
            pltpu.VMEM((N,), jnp.float32),
            pltpu.VMEM((N + 16,), jnp.float32),
            pltpu.VMEM((N + 16,), jnp.int32),
            pltpu.VMEM((N + 16,), jnp.float32),
            pltpu.VMEM((N + 16,), jnp.int32),
            pltpu.VMEM((32 * K_NN,), jnp.float32),
            pltpu.VMEM((32 * K_NN,), jnp.float32),
            pltpu.VMEM((32 * K_NN,), jnp.float32),
            pltpu.SemaphoreType.DMA,
            pltpu.SemaphoreType.DMA,
        ],
    )
    g = run(P.reshape(B * N, 1, N), xyz.reshape(B * 3, 1, N))
    return g.reshape(B, 3, N * K_NN)


def _seg_matrices():
    p = np.arange(PT)
    r = np.arange(RT)
    ind = (p[:, None] // K_NN == r[None, :]).astype(np.float32)
    return jnp.asarray(ind), jnp.asarray(ind.T.copy())


def _dot(a, b):
    return jax.lax.dot_general(a, b, (((1,), (0,)), ((), ())),
                               precision=jax.lax.Precision.HIGHEST,
                               preferred_element_type=jnp.float32)


def _dot_lp(a, b):
    return jax.lax.dot_general(a, b, (((1,), (0,)), ((), ())),
                               preferred_element_type=jnp.float32)


def _pdx_tile(g, q, m_ref, e_ref):
    M = m_ref[...]
    E = e_ref[...]
    inv_k = np.float32(1.0 / K_NN)
    nx = g - q
    means = _dot(_dot(jnp.concatenate([nx, g], axis=0), M) * inv_k, E)
    mean_nx = means[0:3]
    s_nx = jnp.sum(nx * nx, axis=0, keepdims=True)
    pdist1 = jnp.sqrt(s_nx + 1e-12)
    dxm = nx - mean_nx
    pdist2 = jnp.sqrt(jnp.sum(dxm * dxm, axis=0, keepdims=True) + 1e-12)
    norm_meanx = jnp.sqrt(jnp.sum(mean_nx * mean_nx, axis=0, keepdims=True) + 1e-12)
    cos1 = (jnp.sum(nx * mean_nx, axis=0, keepdims=True)
            / jnp.maximum(pdist1 * norm_meanx, 1e-8))
    mean_g = means[3:6]
    norms = g - mean_g
    nnorms = q - mean_g
    pdist3 = jnp.sqrt(4.0 * s_nx + 1e-12)
    norm_norms = jnp.sqrt(jnp.sum(norms * norms, axis=0, keepdims=True) + 1e-12)
    norm_nnorms = jnp.sqrt(jnp.sum(nnorms * nnorms, axis=0, keepdims=True) + 1e-12)
    cos2 = (jnp.sum(nx * norms, axis=0, keepdims=True)
            / jnp.maximum(pdist1 * norm_norms, 1e-8))
    cos3 = (jnp.sum(norms * nnorms, axis=0, keepdims=True)
            / jnp.maximum(norm_norms * norm_nnorms, 1e-8))
    out6 = jnp.concatenate([pdist1, pdist2, pdist3, cos1, cos2, cos3], axis=0)
    mean6 = _dot(_dot(out6, M) * inv_k, E)
    return jnp.concatenate([out6, out6 - mean6], axis=0)


def _gelu(x):
    return 0.5 * x * (1.0 + lax.erf(x * np.float32(1.0 / np.sqrt(2.0))))


def _p1_kernel(g_ref, q_ref, m_ref, e_ref, s_ref, pdx_ref):
    b = pl.program_id(0)
    i = pl.program_id(1)
    pdx = _pdx_tile(g_ref[0], q_ref[0], m_ref, e_ref)
    pdx_ref[0] = pdx
    ones = jnp.ones((1, PT), jnp.float32)
    aug = jnp.concatenate([pdx, ones], axis=0)
    contrib = _dot_lp(aug, aug.T)

    @pl.when(jnp.logical_and(b == 0, i == 0))
    def _():
        s_ref[...] = jnp.zeros_like(s_ref)

    s_ref[...] += contrib


def _p2_kernel(pdx_ref, a1_ref, c1_ref, s_ref):
    b = pl.program_id(0)
    i = pl.program_id(1)
    pdx = pdx_ref[0]
    G = _gelu(_dot(a1_ref[...], pdx) + c1_ref[...])
    ones = jnp.ones((1, PT), jnp.float32)
    aug = jnp.concatenate([G, ones], axis=0)
    contrib = _dot_lp(aug, aug.T)

    @pl.when(jnp.logical_and(b == 0, i == 0))
    def _():
        s_ref[...] = jnp.zeros_like(s_ref)

    s_ref[...] += contrib


def _p3_kernel(pdx_ref, a1_ref, c1_ref, a2_ref, c2_ref, o_ref):
    pdx = pdx_ref[0]
    G = _gelu(_dot(a1_ref[...], pdx) + c1_ref[...])
    h2 = _dot(a2_ref[...], G) + c2_ref[...]
    o_ref[0] = _gelu(h2)


def kernel(x, xyz, W1, g1, b1, W2, bias2, g2, b2):
    B, _, N = xyz.shape
    P = N * K_NN
    ntiles = P // PT

    gs = []
    for bb in range(B):
        xyz_b = lax.slice_in_dim(xyz, bb, bb + 1, axis=0)
        pw_b = _pairwise(xyz_b)
        gs.append(_sc_topk_gather(pw_b, xyz_b))
    g = jnp.concatenate(gs, axis=0)
    xyz_t = jnp.transpose(xyz, (0, 2, 1))
    q = jnp.transpose(jnp.repeat(xyz_t, K_NN, axis=1), (0, 2, 1))

    M, E = _seg_matrices()
    cnt = jnp.float32(B * P)

    f_spec = pl.BlockSpec((1, 3, PT), lambda b, i: (b, 0, i))
    m_spec = pl.BlockSpec((PT, RT), lambda b, i: (0, 0))
    e_spec = pl.BlockSpec((RT, PT), lambda b, i: (0, 0))

    pdx_spec = pl.BlockSpec((1, 12, PT), lambda b, i: (b, 0, i))
    S, PDX = pl.pallas_call(
        _p1_kernel,
        grid=(B, ntiles),
        in_specs=[f_spec, f_spec, m_spec, e_spec],
        out_specs=[pl.BlockSpec((13, 13), lambda b, i: (0, 0)), pdx_spec],
        out_shape=[jax.ShapeDtypeStruct((13, 13), jnp.float32),
                   jax.ShapeDtypeStruct((B, 12, P), jnp.float32)],
    )(g, q, M, E)

    hp = jax.lax.Precision.HIGHEST
    s1 = S[:12, 12] / cnt
    S2n = S[:12, :12] / cnt
    mean_h = jnp.matmul(W1, s1, precision=hp)
    e2 = jnp.sum(jnp.matmul(W1, S2n, precision=hp) * W1, axis=1)
    var1 = e2 - mean_h * mean_h
    a1 = g1 / jnp.sqrt(var1 + 1e-5)
    A1 = a1[:, None] * W1
    c1 = (b1 - a1 * mean_h)[:, None]

    a1_spec = pl.BlockSpec((64, 12), lambda b, i: (0, 0))
    c1_spec = pl.BlockSpec((64, 1), lambda b, i: (0, 0))

    SG = pl.pallas_call(
        _p2_kernel,
        grid=(B, ntiles),
        in_specs=[pdx_spec, a1_spec, c1_spec],
        out_specs=pl.BlockSpec((65, 65), lambda b, i: (0, 0)),
        out_shape=jax.ShapeDtypeStruct((65, 65), jnp.float32),
    )(PDX, A1, c1)

    sg1 = SG[:64, 64] / cnt
    SG2n = SG[:64, :64] / cnt
    w2sg1 = jnp.matmul(W2, sg1, precision=hp)
    mean_h2 = w2sg1 + bias2
    e2b = (jnp.sum(jnp.matmul(W2, SG2n, precision=hp) * W2, axis=1)
           + 2.0 * bias2 * w2sg1 + bias2 * bias2)
    var2 = e2b - mean_h2 * mean_h2
    a2 = g2 / jnp.sqrt(var2 + 1e-5)
    A2 = a2[:, None] * W2
    c2 = (b2 + a2 * (bias2 - mean_h2))[:, None]

    a2_spec = pl.BlockSpec((64, 64), lambda b, i: (0, 0))
    c2_spec = pl.BlockSpec((64, 1), lambda b, i: (0, 0))

    out = pl.pallas_call(
        _p3_kernel,
        grid=(B, ntiles),
        in_specs=[pdx_spec, a1_spec, c1_spec, a2_spec, c2_spec],
        out_specs=pl.BlockSpec((1, 64, PT), lambda b, i: (b, 0, i)),
        out_shape=jax.ShapeDtypeStruct((B, 64, P), jnp.float32),
    )(PDX, A1, c1, A2, c2)

    out = out.reshape(B, 64, N, K_NN)
    return (out, xyz_t)
